# baseline (speedup 1.0000x reference)
import functools

import jax
import jax.numpy as jnp
from jax import lax
from jax.experimental import pallas as pl
from jax.experimental.pallas import tpu as pltpu
from jax.experimental.pallas import tpu_sc as plsc

N, C, D, K = 9216, 192, 64, 1024
F = C + D

BN = 3072
NB = N // BN


def _tc_body(x_ref, we_ref, be_ref, cb_ref, wd_ref, bd_ref,
             ze_ref, idx_ref, tab_ref):
    i = pl.program_id(0)

    cb = cb_ref[...]

    @pl.when(i == 0)
    def _():
        cbdec = (jnp.dot(cb, wd_ref[...], preferred_element_type=jnp.float32)
                 + bd_ref[...])
        tab_ref[...] = jnp.concatenate([cbdec, cb], axis=1)

    x = x_ref[...]
    z = (jnp.dot(x, we_ref[...], preferred_element_type=jnp.float32)
         + be_ref[...])
    ze_ref[...] = z

    scores = lax.dot_general(z, cb, (((1,), (1,)), ((), ())),
                             preferred_element_type=jnp.float32)
    znorm = jnp.sum(z * z, axis=1, keepdims=True)
    cnorm = jnp.sum(cb * cb, axis=1)[None, :]
    d2 = znorm - 2.0 * scores + cnorm
    dist = jnp.sqrt(jnp.clip(d2, 0.0, None))
    idx = jnp.argmin(dist, axis=1).astype(jnp.int32)
    idx_ref[...] = idx


_tc_call = pl.pallas_call(
    _tc_body,
    grid=(NB,),
    in_specs=[
        pl.BlockSpec((BN, C), lambda i: (i, 0)),
        pl.BlockSpec((C, D), lambda i: (0, 0)),
        pl.BlockSpec((1, D), lambda i: (0, 0)),
        pl.BlockSpec((K, D), lambda i: (0, 0)),
        pl.BlockSpec((D, C), lambda i: (0, 0)),
        pl.BlockSpec((1, C), lambda i: (0, 0)),
    ],
    out_specs=[
        pl.BlockSpec((BN, D), lambda i: (i, 0)),
        pl.BlockSpec((BN,), lambda i: (i,)),
        pl.BlockSpec((K, F), lambda i: (0, 0)),
    ],
    out_shape=[
        jax.ShapeDtypeStruct((N, D), jnp.float32),
        jax.ShapeDtypeStruct((N,), jnp.int32),
        jax.ShapeDtypeStruct((K, F), jnp.float32),
    ],
    compiler_params=pltpu.CompilerParams(
        dimension_semantics=("arbitrary",),
    ),
)


_NC, _NS = 2, 16
_NW = _NC * _NS
B_PER_W = N // _NW
NCHUNK = 3
CHUNK = B_PER_W // NCHUNK


@functools.cache
def _sc_gather_call():
    mesh = plsc.VectorSubcoreMesh(
        core_axis_name="c", subcore_axis_name="s", num_cores=_NC)

    @functools.partial(
        pl.kernel,
        mesh=mesh,
        out_type=(
            jax.ShapeDtypeStruct((N, D), jnp.float32),
            jax.ShapeDtypeStruct((N, C), jnp.float32),
        ),
        scratch_types=[
            pltpu.VMEM((B_PER_W,), jnp.int32),
            pltpu.VMEM((B_PER_W, F), jnp.float32),
            pltpu.VMEM((CHUNK, D), jnp.float32),
            pltpu.VMEM((CHUNK, C), jnp.float32),
            pltpu.SemaphoreType.DMA,
            pltpu.SemaphoreType.DMA,
        ],
    )
    def _sc_gather(tab_hbm, idx_hbm, zq_hbm, xh_hbm,
                   idx_v, buf, bufz, bufx, gsem, wsem):
        wid = lax.axis_index("s") * _NC + lax.axis_index("c")
        base = wid * B_PER_W
        pltpu.sync_copy(idx_hbm.at[pl.ds(base, B_PER_W)], idx_v)
        gathers = [
            pltpu.async_copy(
                tab_hbm.at[idx_v.at[pl.ds(j * CHUNK, CHUNK)]],
                buf.at[pl.ds(j * CHUNK, CHUNK)], gsem)
            for j in range(NCHUNK)
        ]
        writes = []
        for j in range(NCHUNK):
            gathers[j].wait()
            if j >= 1:
                writes[2 * (j - 1)].wait()
                writes[2 * (j - 1) + 1].wait()

            def repack_row(r, _, j=j):
                row = j * CHUNK + r
                for k in range(C // 16):
                    bufx[r, pl.ds(16 * k, 16)] = buf[row, pl.ds(16 * k, 16)]
                for k in range(D // 16):
                    bufz[r, pl.ds(16 * k, 16)] = buf[row, pl.ds(C + 16 * k, 16)]
                return _

            lax.fori_loop(0, CHUNK, repack_row, 0)
            rows = pl.ds(base + j * CHUNK, CHUNK)
            writes.append(pltpu.async_copy(bufz, zq_hbm.at[rows], wsem))
            writes.append(pltpu.async_copy(bufx, xh_hbm.at[rows], wsem))
        for w in writes[-2:]:
            w.wait()

    return _sc_gather


def kernel(x, W_enc, b_enc, codebook, W_dec, b_dec):
    z_e, indices, tab = _tc_call(
        x, W_enc, b_enc.reshape(1, D), codebook, W_dec, b_dec.reshape(1, C))
    z_q, x_hat = _sc_gather_call()(tab, indices)
    return (x_hat, z_e, z_q, indices)


# speedup vs baseline: 1.0802x; 1.0802x over previous
import functools

import jax
import jax.numpy as jnp
from jax import lax
from jax.experimental import pallas as pl
from jax.experimental.pallas import tpu as pltpu
from jax.experimental.pallas import tpu_sc as plsc

N, C, D, K = 9216, 192, 64, 1024
F = C + D

BN = 1024
NA = 4096
NB_ROWS = N - NA

_NC, _NS = 2, 16
_NW = _NC * _NS


def _encode_argmin(x, we, be, cb):
    z = (jnp.dot(x, we, preferred_element_type=jnp.float32) + be)
    scores = lax.dot_general(z, cb, (((1,), (1,)), ((), ())),
                             preferred_element_type=jnp.float32)
    znorm = jnp.sum(z * z, axis=1, keepdims=True)
    cnorm = jnp.sum(cb * cb, axis=1)[None, :]
    d2 = znorm - 2.0 * scores + cnorm
    dist = jnp.sqrt(jnp.clip(d2, 0.0, None))
    idx = jnp.argmin(dist, axis=1).astype(jnp.int32)
    return z, idx


def _tc_body_a(x_ref, we_ref, be_ref, cb_ref, wd_ref, bd_ref,
               ze_ref, idx_ref, tab_ref):
    cb = cb_ref[...]

    @pl.when(pl.program_id(0) == 0)
    def _():
        cbdec = (jnp.dot(cb, wd_ref[...], preferred_element_type=jnp.float32)
                 + bd_ref[...])
        tab_ref[...] = jnp.concatenate([cbdec, cb], axis=1)

    z, idx = _encode_argmin(x_ref[...], we_ref[...], be_ref[...], cb)
    ze_ref[...] = z
    idx_ref[...] = idx


def _tc_body_b(x_ref, we_ref, be_ref, cb_ref, ze_ref, idx_ref):
    z, idx = _encode_argmin(x_ref[...], we_ref[...], be_ref[...], cb_ref[...])
    ze_ref[...] = z
    idx_ref[...] = idx


_common_in_specs = [
    pl.BlockSpec((C, D), lambda i: (0, 0)),
    pl.BlockSpec((1, D), lambda i: (0, 0)),
    pl.BlockSpec((K, D), lambda i: (0, 0)),
]

_tc_call_a = pl.pallas_call(
    _tc_body_a,
    grid=(NA // BN,),
    in_specs=[pl.BlockSpec((BN, C), lambda i: (i, 0))] + _common_in_specs + [
        pl.BlockSpec((D, C), lambda i: (0, 0)),
        pl.BlockSpec((1, C), lambda i: (0, 0)),
    ],
    out_specs=[
        pl.BlockSpec((BN, D), lambda i: (i, 0)),
        pl.BlockSpec((BN,), lambda i: (i,)),
        pl.BlockSpec((K, F), lambda i: (0, 0)),
    ],
    out_shape=[
        jax.ShapeDtypeStruct((NA, D), jnp.float32),
        jax.ShapeDtypeStruct((NA,), jnp.int32),
        jax.ShapeDtypeStruct((K, F), jnp.float32),
    ],
    compiler_params=pltpu.CompilerParams(
        dimension_semantics=("arbitrary",),
    ),
)

_tc_call_b = pl.pallas_call(
    _tc_body_b,
    grid=(NB_ROWS // BN,),
    in_specs=[pl.BlockSpec((BN, C), lambda i: (i + NA // BN, 0))]
    + _common_in_specs,
    out_specs=[
        pl.BlockSpec((BN, D), lambda i: (i, 0)),
        pl.BlockSpec((BN,), lambda i: (i,)),
    ],
    out_shape=[
        jax.ShapeDtypeStruct((NB_ROWS, D), jnp.float32),
        jax.ShapeDtypeStruct((NB_ROWS,), jnp.int32),
    ],
    compiler_params=pltpu.CompilerParams(
        dimension_semantics=("arbitrary",),
    ),
)


@functools.cache
def _sc_gather_call(n_rows):
    b_per_w = n_rows // _NW
    nchunk = 1 if b_per_w <= 128 else (2 if b_per_w <= 256 else 3)
    chunk = b_per_w // nchunk
    mesh = plsc.VectorSubcoreMesh(
        core_axis_name="c", subcore_axis_name="s", num_cores=_NC)

    @functools.partial(
        pl.kernel,
        mesh=mesh,
        out_type=jax.ShapeDtypeStruct((n_rows, F), jnp.float32),
        scratch_types=[
            pltpu.VMEM((b_per_w,), jnp.int32),
            pltpu.VMEM((b_per_w, F), jnp.float32),
            pltpu.SemaphoreType.DMA,
            pltpu.SemaphoreType.DMA,
        ],
    )
    def _sc_gather(tab_hbm, idx_hbm, out_hbm, idx_v, buf, gsem, wsem):
        wid = lax.axis_index("s") * _NC + lax.axis_index("c")
        base = wid * b_per_w
        pltpu.sync_copy(idx_hbm.at[pl.ds(base, b_per_w)], idx_v)
        gathers = [
            pltpu.async_copy(
                tab_hbm.at[idx_v.at[pl.ds(j * chunk, chunk)]],
                buf.at[pl.ds(j * chunk, chunk)], gsem)
            for j in range(nchunk)
        ]
        writes = []
        for j in range(nchunk):
            gathers[j].wait()
            rows = pl.ds(j * chunk, chunk)
            writes.append(pltpu.async_copy(
                buf.at[rows],
                out_hbm.at[pl.ds(base + j * chunk, chunk)], wsem))
        for w in writes:
            w.wait()

    return _sc_gather


def kernel(x, W_enc, b_enc, codebook, W_dec, b_dec):
    be = b_enc.reshape(1, D)
    z_e_a, idx_a, tab = _tc_call_a(
        x, W_enc, be, codebook, W_dec, b_dec.reshape(1, C))
    out_a = _sc_gather_call(NA)(tab, idx_a)
    z_e_b, idx_b = _tc_call_b(x, W_enc, be, codebook)
    out_b = _sc_gather_call(NB_ROWS)(tab, idx_b)
    z_e = jnp.concatenate([z_e_a, z_e_b], axis=0)
    indices = jnp.concatenate([idx_a, idx_b], axis=0)
    x_hat = jnp.concatenate([out_a[:, :C], out_b[:, :C]], axis=0)
    z_q = jnp.concatenate([out_a[:, C:], out_b[:, C:]], axis=0)
    return (x_hat, z_e, z_q, indices)
